# baseline (speedup 1.0000x reference)
import jax
import jax.numpy as jnp
from jax import lax
from jax.experimental import pallas as pl
from jax.experimental.pallas import tpu as pltpu
from jax.experimental.pallas import tpu_sc as plsc

B = 16384
F = 26
D = 32
NW = 32
ROWS = B // NW
ELEMS = ROWS * F
GCH = 128
NCH = ELEMS // GCH
LOOKAHEAD = 8


CPF = ROWS // GCH


def _first_order_body(idx_hbm, val_hbm, w_hbm, out_hbm, idx_v, val_v, g_v,
                      out_v, sem):
    wid = lax.axis_index("s") * 2 + lax.axis_index("c")
    base = wid * ROWS
    pltpu.sync_copy(idx_hbm.at[:, pl.ds(base, ROWS)], idx_v)
    pltpu.sync_copy(val_hbm.at[:, pl.ds(base, ROWS)], val_v)

    @pl.loop(0, NCH + LOOKAHEAD)
    def _pipe(c):
        @pl.when(c < NCH)
        def _fire():
            f, off = c // CPF, (c % CPF) * GCH
            sl = pl.ds(off, GCH)
            pltpu.async_copy(w_hbm.at[idx_v.at[f, sl]], g_v.at[f, sl], sem)

        @pl.when(c >= LOOKAHEAD)
        def _drain():
            d = c - LOOKAHEAD
            f, off = d // CPF, (d % CPF) * GCH
            sl = pl.ds(off, GCH)
            pltpu.make_async_copy(w_hbm.at[idx_v.at[f, sl]], g_v.at[f, sl],
                                  sem).wait()

    @pl.loop(0, ROWS // 16)
    def _rows(j):
        sl = pl.ds(j * 16, 16)
        acc = jnp.zeros((16,), jnp.float32)
        for f in range(F):
            acc += g_v[f, sl] * val_v[f, sl]
        out_v[sl] = acc

    pltpu.sync_copy(out_v, out_hbm.at[pl.ds(base, ROWS)])


def _first_order(idx_t, val_t, w_flat):
    mesh = plsc.VectorSubcoreMesh(core_axis_name="c", subcore_axis_name="s")
    return pl.kernel(
        _first_order_body,
        out_type=jax.ShapeDtypeStruct((B,), jnp.float32),
        mesh=mesh,
        scratch_types=[
            pltpu.VMEM((F, ROWS), jnp.int32),
            pltpu.VMEM((F, ROWS), jnp.float32),
            pltpu.VMEM((F, ROWS), jnp.float32),
            pltpu.VMEM((ROWS,), jnp.float32),
            pltpu.SemaphoreType.DMA,
        ],
    )(idx_t, val_t, w_flat)


BB = 1024


def _second_order_body(x_ref, o_ref):
    x = x_ref[...]
    row = lax.broadcasted_iota(jnp.int32, (F * D, D), 0)
    col = lax.broadcasted_iota(jnp.int32, (F * D, D), 1)
    m = (row % D == col).astype(jnp.float32)
    s = lax.dot_general(x, m, (((1,), (0,)), ((), ())),
                        preferred_element_type=jnp.float32)
    ss = jnp.sum(s * s, axis=1, keepdims=True)
    sq = jnp.sum(x * x, axis=1, keepdims=True)
    o_ref[...] = 0.5 * (ss - sq)


def _second_order(x2d):
    return pl.pallas_call(
        _second_order_body,
        grid=(B // BB,),
        in_specs=[pl.BlockSpec((BB, F * D), lambda i: (i, 0))],
        out_specs=pl.BlockSpec((BB, 1), lambda i: (i, 0)),
        out_shape=jax.ShapeDtypeStruct((B, 1), jnp.float32),
        compiler_params=pltpu.CompilerParams(
            dimension_semantics=("arbitrary",)),
    )(x2d)


def kernel(inputs_index, inputs_value, embed_inputs, w):
    idx_t = inputs_index.T.astype(jnp.int32)
    val_t = inputs_value.T
    w_flat = w.reshape(-1)
    first = _first_order(idx_t, val_t, w_flat)
    second = _second_order(embed_inputs.reshape(B, F * D))
    return first[:, None] + second


# speedup vs baseline: 1.7373x; 1.7373x over previous
import jax
import jax.numpy as jnp
from jax import lax
from jax.experimental import pallas as pl
from jax.experimental.pallas import tpu as pltpu
from jax.experimental.pallas import tpu_sc as plsc

B = 16384
F = 26
D = 32
NW = 32
ROWS = B // NW
GCH = 128
CPF = ROWS // GCH
NCH = F * CPF
LOOKAHEAD = 16


def _first_order_body(idx_hbm, val_hbm, w_hbm, out_hbm, idx_v, val_v, g_v,
                      out_v, sem):
    wid = lax.axis_index("s") * 2 + lax.axis_index("c")
    base = wid * ROWS
    pltpu.sync_copy(idx_hbm.at[:, pl.ds(base, ROWS)], idx_v)
    pltpu.sync_copy(val_hbm.at[:, pl.ds(base, ROWS)], val_v)

    @pl.loop(0, NCH + LOOKAHEAD)
    def _pipe(c):
        @pl.when(c < NCH)
        def _fire():
            f, off = c // CPF, (c % CPF) * GCH
            sl = pl.ds(off, GCH)
            pltpu.async_copy(w_hbm.at[idx_v.at[f, sl]], g_v.at[f, sl], sem)

        @pl.when(c >= LOOKAHEAD)
        def _drain():
            d = c - LOOKAHEAD
            f, off = d // CPF, (d % CPF) * GCH
            sl = pl.ds(off, GCH)
            pltpu.make_async_copy(w_hbm.at[idx_v.at[f, sl]], g_v.at[f, sl],
                                  sem).wait()

    @pl.loop(0, ROWS // 16)
    def _rows(j):
        sl = pl.ds(j * 16, 16)
        acc = jnp.zeros((16,), jnp.float32)
        for f in range(F):
            acc += g_v[f, sl] * val_v[f, sl]
        out_v[sl] = acc

    pltpu.sync_copy(out_v, out_hbm.at[pl.ds(base, ROWS)])


def _first_order(idx_t, val_t, w_flat):
    mesh = plsc.VectorSubcoreMesh(core_axis_name="c", subcore_axis_name="s")
    return pl.kernel(
        _first_order_body,
        out_type=jax.ShapeDtypeStruct((B,), jnp.float32),
        mesh=mesh,
        scratch_types=[
            pltpu.VMEM((F, ROWS), jnp.int32),
            pltpu.VMEM((F, ROWS), jnp.float32),
            pltpu.VMEM((F, ROWS), jnp.float32),
            pltpu.VMEM((ROWS,), jnp.float32),
            pltpu.SemaphoreType.DMA,
        ],
    )(idx_t, val_t, w_flat)


CBB = 2048


def _second_order_body(x_ref, o_ref):
    x = x_ref[...]
    sq = jnp.sum(x * x, axis=0)
    s = jnp.zeros((D, CBB), jnp.float32)
    for f in range(F):
        s += x[f * D:(f + 1) * D, :]
    ss = jnp.sum(s * s, axis=0)
    o_ref[...] = 0.5 * (ss - sq)


def _second_order(x2d):
    return pl.pallas_call(
        _second_order_body,
        grid=(B // CBB,),
        in_specs=[pl.BlockSpec((F * D, CBB), lambda i: (0, i))],
        out_specs=pl.BlockSpec((CBB,), lambda i: (i,)),
        out_shape=jax.ShapeDtypeStruct((B,), jnp.float32),
        compiler_params=pltpu.CompilerParams(
            dimension_semantics=("arbitrary",)),
    )(x2d)


def kernel(inputs_index, inputs_value, embed_inputs, w):
    idx_t = inputs_index.T.astype(jnp.int32)
    val_t = inputs_value.T
    x2d = jnp.transpose(embed_inputs, (1, 2, 0)).reshape(F * D, B)
    w_flat = w.reshape(-1)
    first = _first_order(idx_t, val_t, w_flat)
    second = _second_order(x2d)
    return (first + second)[:, None]
